# baseline (speedup 1.0000x reference)
import functools

import jax
import jax.numpy as jnp
from jax import lax
from jax.experimental import pallas as pl
from jax.experimental.pallas import tpu as pltpu
from jax.experimental.pallas import tpu_sc as plsc

N = 2_000_000
L = 16
NC = 2
NS = 16
NW = NC * NS
NBINS = 256
CHUNK = 4464
CPW = 14
TAIL = N - NW * CPW * CHUNK
TAIL_VREGS = TAIL // L

_MAGIC = 0x5F3759DF


def _radius_and_tid(x, y, c00, c01, c10, c11):
    trace = c00 + c11
    det = c00 * c11 - c01 * c10
    s = jnp.maximum(trace * trace - 4.0 * det, 0.0)
    yi = _MAGIC - lax.shift_right_arithmetic(plsc.bitcast(s, jnp.int32), 1)
    yr = plsc.bitcast(yi, jnp.float32)
    h = 0.5 * s
    yr = yr * (1.5 - h * yr * yr)
    yr = yr * (1.5 - h * yr * yr)
    yr = yr * (1.5 - h * yr * yr)
    sq = s * yr
    radius = 0.5 * trace + 0.5 * sq
    ix = (x * 16.0).astype(jnp.int32)
    iy = (y * 16.0).astype(jnp.int32)
    tx = jnp.minimum(jnp.maximum(ix, 0), 15)
    ty = jnp.minimum(jnp.maximum(iy, 0), 15)
    tid = ty * 16 + tx
    return radius, tid


def _stage1_body(pos_h, cov_h, opa_h, rad_h, cnt_p, wgt_p,
                 pos_b, cov_b, opa_b, rad_b, hist_c, hist_w, row_c, row_w,
                 isem, osem):
    wid = lax.axis_index("s") * NC + lax.axis_index("c")
    iota = lax.iota(jnp.int32, L)
    iota2 = iota * 2
    iota4 = iota * 4
    laneoff = iota * NBINS
    ones = jnp.full((L,), 1, jnp.int32)
    zi = jnp.zeros((L,), jnp.int32)
    zf = jnp.zeros((L,), jnp.float32)

    def zbody(i, _):
        hist_c[pl.ds(i * L, L)] = zi
        hist_w[pl.ds(i * L, L)] = zf
        return 0
    lax.fori_loop(0, NBINS, zbody, 0)

    def compute(nv):
        def vbody(v, _):
            b16 = v * L
            idx_x = b16 * 2 + iota2
            x = plsc.load_gather(pos_b, [idx_x])
            y = plsc.load_gather(pos_b, [idx_x + 1])
            idx_c = b16 * 4 + iota4
            c00 = plsc.load_gather(cov_b, [idx_c])
            c01 = plsc.load_gather(cov_b, [idx_c + 1])
            c10 = plsc.load_gather(cov_b, [idx_c + 2])
            c11 = plsc.load_gather(cov_b, [idx_c + 3])
            opa = opa_b[pl.ds(b16, L)]
            radius, tid = _radius_and_tid(x, y, c00, c01, c10, c11)
            rad_b[pl.ds(b16, L)] = radius
            hidx = laneoff + tid
            plsc.addupdate_scatter(hist_c, [hidx], ones)
            plsc.addupdate_scatter(hist_w, [hidx], opa * radius)
            return 0
        lax.fori_loop(0, nv, vbody, 0)

    def cbody(i, _):
        gc = wid * CPW + i
        pltpu.sync_copy(pos_h.at[pl.ds(gc * 2 * CHUNK, 2 * CHUNK)], pos_b)
        pltpu.sync_copy(cov_h.at[pl.ds(gc * 4 * CHUNK, 4 * CHUNK)], cov_b)
        pltpu.sync_copy(opa_h.at[pl.ds(gc * CHUNK, CHUNK)], opa_b)
        compute(CHUNK // L)
        pltpu.sync_copy(rad_b, rad_h.at[pl.ds(gc * CHUNK, CHUNK)])
        return 0
    lax.fori_loop(0, CPW, cbody, 0)

    @pl.when(wid == NW - 1)
    def _tail():
        base = NW * CPW * CHUNK
        pltpu.sync_copy(pos_h.at[pl.ds(base * 2, 2 * TAIL)],
                        pos_b.at[pl.ds(0, 2 * TAIL)])
        pltpu.sync_copy(cov_h.at[pl.ds(base * 4, 4 * TAIL)],
                        cov_b.at[pl.ds(0, 4 * TAIL)])
        pltpu.sync_copy(opa_h.at[pl.ds(base, TAIL)],
                        opa_b.at[pl.ds(0, TAIL)])
        compute(TAIL_VREGS)
        pltpu.sync_copy(rad_b.at[pl.ds(0, TAIL)],
                        rad_h.at[pl.ds(base, TAIL)])

    def rbody(g, _):
        g16 = g * L
        acc_c = zi
        acc_w = zf
        for l in range(L):
            acc_c = acc_c + hist_c[pl.ds(l * NBINS + g16, L)]
            acc_w = acc_w + hist_w[pl.ds(l * NBINS + g16, L)]
        row_c[pl.ds(g16, L)] = acc_c
        row_w[pl.ds(g16, L)] = acc_w
        return 0
    lax.fori_loop(0, NBINS // L, rbody, 0)

    pltpu.sync_copy(row_c, cnt_p.at[pl.ds(wid * NBINS, NBINS)])
    pltpu.sync_copy(row_w, wgt_p.at[pl.ds(wid * NBINS, NBINS)])


def _stage2_body(cnt_p, wgt_p, offs_h, wcum_h, buf_c, buf_w, row_c, row_w):
    wid = lax.axis_index("s") * NC + lax.axis_index("c")

    @pl.when(wid == 0)
    def _merge():
        pltpu.sync_copy(cnt_p, buf_c)
        pltpu.sync_copy(wgt_p, buf_w)
        carry_c = jnp.int32(0)
        carry_w = jnp.float32(0.0)
        for g in range(NBINS // L):
            g16 = g * L
            acc_c = jnp.zeros((L,), jnp.int32)
            acc_w = jnp.zeros((L,), jnp.float32)
            for r in range(NW):
                acc_c = acc_c + buf_c[pl.ds(r * NBINS + g16, L)]
                acc_w = acc_w + buf_w[pl.ds(r * NBINS + g16, L)]
            row_c[pl.ds(g16, L)] = plsc.cumsum(acc_c) + carry_c
            row_w[pl.ds(g16, L)] = plsc.cumsum(acc_w) + carry_w
            carry_c = carry_c + jnp.sum(acc_c)
            carry_w = carry_w + jnp.sum(acc_w)
        pltpu.sync_copy(row_c, offs_h)
        pltpu.sync_copy(row_w, wcum_h)


_MESH = plsc.VectorSubcoreMesh(core_axis_name="c", subcore_axis_name="s",
                               num_cores=NC, num_subcores=NS)

_stage1 = pl.kernel(
    _stage1_body,
    out_type=(
        jax.ShapeDtypeStruct((N,), jnp.float32),
        jax.ShapeDtypeStruct((NW * NBINS,), jnp.int32),
        jax.ShapeDtypeStruct((NW * NBINS,), jnp.float32),
    ),
    mesh=_MESH,
    scratch_types=[
        pltpu.VMEM((2 * CHUNK,), jnp.float32),
        pltpu.VMEM((4 * CHUNK,), jnp.float32),
        pltpu.VMEM((CHUNK,), jnp.float32),
        pltpu.VMEM((CHUNK,), jnp.float32),
        pltpu.VMEM((L * NBINS,), jnp.int32),
        pltpu.VMEM((L * NBINS,), jnp.float32),
        pltpu.VMEM((NBINS,), jnp.int32),
        pltpu.VMEM((NBINS,), jnp.float32),
        pltpu.SemaphoreType.DMA,
        pltpu.SemaphoreType.DMA,
    ],
    compiler_params=pltpu.CompilerParams(needs_layout_passes=False),
    name="gaussian_render_stage1",
)

_stage2 = pl.kernel(
    _stage2_body,
    out_type=(
        jax.ShapeDtypeStruct((NBINS,), jnp.int32),
        jax.ShapeDtypeStruct((NBINS,), jnp.float32),
    ),
    mesh=_MESH,
    scratch_types=[
        pltpu.VMEM((NW * NBINS,), jnp.int32),
        pltpu.VMEM((NW * NBINS,), jnp.float32),
        pltpu.VMEM((NBINS,), jnp.int32),
        pltpu.VMEM((NBINS,), jnp.float32),
    ],
    compiler_params=pltpu.CompilerParams(needs_layout_passes=False),
    name="gaussian_render_stage2",
)


@jax.jit
def _run(pos_flat, cov_flat, opacity):
    radius, cnt_p, wgt_p = _stage1(pos_flat, cov_flat, opacity)
    tile_offsets, tile_weight_cum = _stage2(cnt_p, wgt_p)
    return radius, tile_offsets, tile_weight_cum


def kernel(pos2d, cov2d, opacity, num_tile):
    del num_tile
    assert pos2d.shape[0] == N
    return _run(pos2d.reshape(-1), cov2d.reshape(-1), opacity)
